# baseline (speedup 1.0000x reference)
import functools

import jax
import jax.numpy as jnp
from jax import lax
from jax.experimental import pallas as pl
from jax.experimental.pallas import tpu as pltpu
from jax.experimental.pallas import tpu_sc as plsc

L = 16
C = 80
G = C // L

_PERM_DNUMS = lax.GatherDimensionNumbers(
    offset_dims=(), collapsed_slice_dims=(0,), start_index_map=(0,))


def _vperm(v, idx):
    return lax.gather(v, idx[:, None], _PERM_DNUMS, (1,),
                      mode=lax.GatherScatterMode.PROMISE_IN_BOUNDS)


def _prep_body(x_ref, wq_ref, w1_ref, qb_ref, q_ref, c_ref):
    x = x_ref[...]
    a = jnp.dot(wq_ref[...].T, w1_ref[...], preferred_element_type=jnp.float32)
    q = jnp.dot(x, a, preferred_element_type=jnp.float32)
    q = q + jnp.dot(qb_ref[...], w1_ref[...], preferred_element_type=jnp.float32)
    q_ref[...] = q
    d = x.shape[1]
    xn2 = jnp.sum(x * x, axis=1, keepdims=True)
    qn2 = jnp.sum(q * q, axis=1, keepdims=True)
    c_ref[...] = jnp.sqrt(qn2) * (jnp.sqrt(jnp.max(xn2)) / (d ** 0.5))


def _xw_body(x_ref, w_ref, o_ref):
    o_ref[0] = jnp.dot(x_ref[...], w_ref[0], preferred_element_type=jnp.float32)


def _rden_body(d_ref, r_ref):
    r_ref[...] = 1.0 / (d_ref[0:1, :] + d_ref[1:2, :])


def _final_body(p_ref, b_ref, g_ref, bb_ref, o_ref):
    o = p_ref[0] + p_ref[1] + b_ref[...]
    m = jnp.mean(o, axis=1, keepdims=True)
    ctr = o - m
    v = jnp.mean(ctr * ctr, axis=1, keepdims=True)
    o = ctr * jax.lax.rsqrt(v + 1e-5) * g_ref[...] + bb_ref[...]
    o_ref[...] = jnp.maximum(o, 0.0)


P1_SLOTS = 4


def _sc_pass1(n, e, rn, nc, ns):
    epw = e // (nc * ns)
    nchunks = epw // C
    seg_per_tile = rn // ns
    mesh = plsc.VectorSubcoreMesh(core_axis_name="c", subcore_axis_name="s")
    inv_scale = 1.0 / (128.0 ** 0.5)

    def body(src_h, dst_h, et_h, q_h, x_h, c_h, ex_h, den_h, *scr):
        srcb, dstb, etb, cloc, zbuf = scr[0:5]
        qrows = scr[5:7]
        xrows = scr[7:9]
        srcc = scr[9:13]
        dstc = scr[13:17]
        segb = scr[17:21]
        exb = scr[21:25]
        den_sh = scr[25]
        gsem = scr[26:28]
        slin = scr[28:30]
        sind = scr[30:32]

        cid = lax.axis_index("c")
        sid = lax.axis_index("s")
        wid = cid * ns + sid
        base = wid * epw
        myseg = pl.ds(sid * seg_per_tile, seg_per_tile)

        def zf(i, carry):
            zbuf[pl.ds(i * L, L)] = jnp.zeros((L,), jnp.float32)
            return carry

        lax.fori_loop(0, seg_per_tile // L, zf, 0)
        pltpu.sync_copy(zbuf, den_sh.at[myseg])
        pltpu.sync_copy(src_h.at[pl.ds(base, epw)], srcb)
        pltpu.sync_copy(dst_h.at[pl.ds(base, epw)], dstb)
        pltpu.sync_copy(et_h.at[pl.ds(base, epw)], etb)
        pltpu.sync_copy(c_h, cloc)
        plsc.subcore_barrier()

        lanes = lax.iota(jnp.int32, L)
        strides = (8, 4, 2, 1)
        perms = {sh: lanes ^ sh for sh in strides}
        masks = {sh: (lanes & sh) == 0 for sh in strides}

        def stage(ci, s):
            off = ci * C

            def stg(g, carry):
                sl = pl.ds(off + g * L, L)
                osl = pl.ds(g * L, L)
                d16 = dstb[sl]
                srcc[s][osl] = srcb[sl]
                dstc[s][osl] = d16
                segb[s][osl] = etb[sl] * n + d16
                return carry

            lax.fori_loop(0, G, stg, 0, unroll=True)

        def issue_gather(s, rb):
            pltpu.async_copy(q_h.at[dstc[s]], qrows[rb], gsem[rb])
            pltpu.async_copy(x_h.at[srcc[s]], xrows[rb], gsem[rb])

        def wait_gather(s, rb):
            pltpu.make_async_copy(q_h.at[dstc[s]], qrows[rb], gsem[rb]).wait()
            pltpu.make_async_copy(x_h.at[srcc[s]], xrows[rb], gsem[rb]).wait()

        def compute(s, rb):
            qr = qrows[rb]
            xr = xrows[rb]

            def dot(g, carry):
                osl = pl.ds(g * L, L)
                accs = []
                for j in range(L):
                    row = g * L + j
                    acc = qr[row, pl.ds(0, L)] * xr[row, pl.ds(0, L)]
                    for k in range(1, 8):
                        sl2 = pl.ds(k * L, L)
                        acc = acc + qr[row, sl2] * xr[row, sl2]
                    accs.append(acc)
                for sh in strides:
                    half = len(accs) // 2
                    accs = [
                        jnp.where(masks[sh],
                                  accs[j] + _vperm(accs[j], perms[sh]),
                                  accs[j + half]
                                  + _vperm(accs[j + half], perms[sh]))
                        for j in range(half)
                    ]
                c16 = plsc.load_gather(cloc, [dstc[s][osl]])
                exb[s][osl] = jnp.exp(accs[0] * inv_scale - c16)
                return carry

            lax.fori_loop(0, G, dot, 0)

        def issue_stores(ci, s, rb):
            pltpu.async_copy(exb[s], ex_h.at[pl.ds(base + ci * C, C)],
                             slin[rb])
            pltpu.async_copy(exb[s], den_sh.at[segb[s]], sind[rb], add=True)

        def drain_stores(ci, s, rb):
            pltpu.make_async_copy(exb[s], ex_h.at[pl.ds(base + ci * C, C)],
                                  slin[rb]).wait()
            pltpu.make_async_copy(exb[s], den_sh.at[segb[s]],
                                  sind[rb]).wait()

        stage(0, 0)
        issue_gather(0, 0)
        stage(1, 1)
        issue_gather(1, 1)

        nb_outer = (nchunks + P1_SLOTS - 1) // P1_SLOTS

        def outer(h, carry):
            for b in range(P1_SLOTS):
                rb = b & 1
                s2 = (b + 2) % P1_SLOTS
                ci = h * P1_SLOTS + b

                @pl.when(ci < nchunks)
                def _():
                    wait_gather(b, rb)
                    compute(b, rb)

                    @pl.when(ci >= 2)
                    def _():
                        drain_stores(ci - 2, s2, rb)

                    issue_stores(ci, b, rb)

                    @pl.when(ci + 2 < nchunks)
                    def _():
                        stage(ci + 2, s2)
                        issue_gather(s2, rb)

            return carry

        lax.fori_loop(0, nb_outer, outer, 0)

        drain_stores(nchunks - 2, (nchunks - 2) % P1_SLOTS, (nchunks - 2) & 1)
        drain_stores(nchunks - 1, (nchunks - 1) % P1_SLOTS, (nchunks - 1) & 1)

        plsc.subcore_barrier()

        pltpu.sync_copy(den_sh.at[myseg], zbuf)
        pltpu.sync_copy(zbuf,
                        den_h.at[pl.ds(cid * rn + sid * seg_per_tile,
                                       seg_per_tile)])

    return pl.kernel(
        body,
        mesh=mesh,
        compiler_params=pltpu.CompilerParams(needs_layout_passes=False),
        out_type=[
            jax.ShapeDtypeStruct((e,), jnp.float32),
            jax.ShapeDtypeStruct((nc * rn,), jnp.float32),
        ],
        scratch_types=(
            [
                pltpu.VMEM((epw,), jnp.int32),
                pltpu.VMEM((epw,), jnp.int32),
                pltpu.VMEM((epw,), jnp.int32),
                pltpu.VMEM((n,), jnp.float32),
                pltpu.VMEM((rn // ns,), jnp.float32),
            ]
            + [pltpu.VMEM((C, 128), jnp.float32)] * 4
            + [pltpu.VMEM((C,), jnp.int32)] * (3 * P1_SLOTS)
            + [pltpu.VMEM((C,), jnp.float32)] * P1_SLOTS
            + [pltpu.VMEM_SHARED((rn,), jnp.float32)]
            + [pltpu.SemaphoreType.DMA] * 6
        ),
    )


P2_SLOTS = 6


def _sc_pass2(n, e, rn, nc, ns):
    epw = e // (nc * ns)
    nchunks = epw // C
    mesh = plsc.VectorSubcoreMesh(core_axis_name="c", subcore_axis_name="s")
    nblk = n // C

    def body(src_h, dst_h, et_h, ex_h, xw_h, rden_h, outp_h, *scr):
        wb = scr[0]
        rowsa = scr[1:3]
        rowsb = scr[3:5]
        srcc = scr[5:11]
        dstc = scr[11:17]
        etc_ = scr[17:23]
        excb = scr[23:29]
        gsrcc = scr[29:35]
        segc = scr[35:41]
        rdenc = scr[41:47]
        out_sh = scr[47]
        gsem = scr[48:50]
        scsem = scr[50:52]
        lsem = scr[52:58]

        cid = lax.axis_index("c")
        sid = lax.axis_index("s")
        wid = cid * ns + sid
        base = wid * epw

        def zrow(i, carry):
            for k in range(8):
                rowsa[0][i, pl.ds(k * L, L)] = jnp.zeros((L,), jnp.float32)
            return carry

        lax.fori_loop(0, C, zrow, 0)

        def zblk(i, carry):
            blk = sid + i * ns

            @pl.when(blk < nblk)
            def _():
                pltpu.sync_copy(rowsa[0], out_sh.at[pl.ds(blk * C, C)])

            return carry

        lax.fori_loop(0, (nblk + ns - 1) // ns, zblk, 0)
        plsc.subcore_barrier()

        def issue_loads(ci, s):
            gb = base + ci * C
            pltpu.async_copy(src_h.at[pl.ds(gb, C)], srcc[s], lsem[s])
            pltpu.async_copy(dst_h.at[pl.ds(gb, C)], dstc[s], lsem[s])
            pltpu.async_copy(et_h.at[pl.ds(gb, C)], etc_[s], lsem[s])
            pltpu.async_copy(ex_h.at[pl.ds(gb, C)], excb[s], lsem[s])

        def drain_loads(ci, s):
            gb = base + ci * C
            pltpu.make_async_copy(src_h.at[pl.ds(gb, C)], srcc[s],
                                  lsem[s]).wait()
            pltpu.make_async_copy(dst_h.at[pl.ds(gb, C)], dstc[s],
                                  lsem[s]).wait()
            pltpu.make_async_copy(et_h.at[pl.ds(gb, C)], etc_[s],
                                  lsem[s]).wait()
            pltpu.make_async_copy(ex_h.at[pl.ds(gb, C)], excb[s],
                                  lsem[s]).wait()

        def stage(s):
            def stg(g, carry):
                osl = pl.ds(g * L, L)
                gsrcc[s][osl] = etc_[s][osl] * n + srcc[s][osl]
                segc[s][osl] = etc_[s][osl] * n + dstc[s][osl]
                return carry

            lax.fori_loop(0, G, stg, 0, unroll=True)

        def issue_gather(s, rb):
            pltpu.async_copy(xw_h.at[gsrcc[s]], rowsa[rb], gsem[rb])
            pltpu.async_copy(rden_h.at[segc[s]], rdenc[s], gsem[rb])

        def wait_gather(s, rb):
            pltpu.make_async_copy(xw_h.at[gsrcc[s]], rowsa[rb],
                                  gsem[rb]).wait()
            pltpu.make_async_copy(rden_h.at[segc[s]], rdenc[s],
                                  gsem[rb]).wait()

        def compute(s, rb):
            ra = rowsa[rb]
            rb_ = rowsb[rb]

            def scale(g, carry):
                osl = pl.ds(g * L, L)
                wb[osl] = excb[s][osl] * rdenc[s][osl]
                for j in range(L):
                    row = g * L + j
                    wj = plsc.load_gather(wb, [jnp.full((L,), row, jnp.int32)])
                    for k in range(8):
                        sl2 = pl.ds(k * L, L)
                        rb_[row, sl2] = ra[row, sl2] * wj
                return carry

            lax.fori_loop(0, G, scale, 0)

        def issue_scatter(s, rb):
            pltpu.async_copy(rowsb[rb], out_sh.at[dstc[s]], scsem[rb],
                             add=True)

        def drain_scatter(s, rb):
            pltpu.make_async_copy(rowsb[rb], out_sh.at[dstc[s]],
                                  scsem[rb]).wait()

        for s in range(4):
            issue_loads(s, s)
        drain_loads(0, 0)
        stage(0)
        issue_gather(0, 0)
        drain_loads(1, 1)
        stage(1)
        issue_gather(1, 1)

        nb_outer = (nchunks + P2_SLOTS - 1) // P2_SLOTS

        def outer(h, carry):
            for b in range(P2_SLOTS):
                rb = b & 1
                s2 = (b + 2) % P2_SLOTS
                s4 = (b + 4) % P2_SLOTS
                ci = h * P2_SLOTS + b

                @pl.when(ci < nchunks)
                def _():
                    wait_gather(b, rb)

                    @pl.when(ci >= 2)
                    def _():
                        drain_scatter(s4, rb)

                    compute(b, rb)
                    issue_scatter(b, rb)

                    @pl.when(ci + 4 < nchunks)
                    def _():
                        issue_loads(ci + 4, s4)

                    @pl.when(ci + 2 < nchunks)
                    def _():
                        drain_loads(ci + 2, s2)
                        stage(s2)
                        issue_gather(s2, rb)

            return carry

        lax.fori_loop(0, nb_outer, outer, 0)

        drain_scatter((nchunks - 2) % P2_SLOTS, (nchunks - 2) & 1)
        drain_scatter((nchunks - 1) % P2_SLOTS, (nchunks - 1) & 1)

        plsc.subcore_barrier()

        def dblk(i, carry):
            blk = sid + i * ns

            @pl.when(blk < nblk)
            def _():
                pltpu.sync_copy(out_sh.at[pl.ds(blk * C, C)], rowsa[0])
                pltpu.sync_copy(rowsa[0], outp_h.at[cid, pl.ds(blk * C, C)])

            return carry

        lax.fori_loop(0, (nblk + ns - 1) // ns, dblk, 0)

    return pl.kernel(
        body,
        mesh=mesh,
        compiler_params=pltpu.CompilerParams(needs_layout_passes=False),
        out_type=jax.ShapeDtypeStruct((nc, n, 128), jnp.float32),
        scratch_types=(
            [pltpu.VMEM((C,), jnp.float32)]
            + [pltpu.VMEM((C, 128), jnp.float32)] * 4
            + [pltpu.VMEM((C,), jnp.int32)] * (3 * P2_SLOTS)
            + [pltpu.VMEM((C,), jnp.float32)] * P2_SLOTS
            + [pltpu.VMEM((C,), jnp.int32)] * (2 * P2_SLOTS)
            + [pltpu.VMEM((C,), jnp.float32)] * P2_SLOTS
            + [pltpu.VMEM_SHARED((n, 128), jnp.float32)]
            + [pltpu.SemaphoreType.DMA] * (4 + P2_SLOTS)
        ),
    )


@jax.jit
def kernel(x, edge_index, edge_type, rule_ids, W_r, Wq_w, Wq_b, Wk_w, Wk_b,
           rule_emb, bias, ln_g, ln_b):
    n, d = x.shape
    e = edge_type.shape[0]
    r = W_r.shape[0]
    rn = r * n
    info = plsc.get_sparse_core_info()
    nc, ns = info.num_cores, info.num_subcores

    src = edge_index[0]
    dst = edge_index[1]
    et = edge_type.astype(jnp.int32)
    w1 = Wk_w[:, :d]

    q, c2 = pl.pallas_call(
        _prep_body,
        out_shape=[
            jax.ShapeDtypeStruct((n, d), jnp.float32),
            jax.ShapeDtypeStruct((n, 1), jnp.float32),
        ],
    )(x, Wq_w, w1, Wq_b.reshape(1, d))
    c = c2.reshape(n)

    xw = pl.pallas_call(
        _xw_body,
        grid=(r,),
        in_specs=[
            pl.BlockSpec((n, d), lambda i: (0, 0)),
            pl.BlockSpec((1, d, d), lambda i: (i, 0, 0)),
        ],
        out_specs=pl.BlockSpec((1, n, d), lambda i: (i, 0, 0)),
        out_shape=jax.ShapeDtypeStruct((r, n, d), jnp.float32),
    )(x, W_r).reshape(rn, d)

    ex, den = _sc_pass1(n, e, rn, nc, ns)(src, dst, et, q, x, c)

    rden = pl.pallas_call(
        _rden_body,
        out_shape=jax.ShapeDtypeStruct((1, rn), jnp.float32),
    )(den.reshape(nc, rn)).reshape(rn)

    outp = _sc_pass2(n, e, rn, nc, ns)(src, dst, et, ex, xw, rden)

    out = pl.pallas_call(
        _final_body,
        out_shape=jax.ShapeDtypeStruct((n, d), jnp.float32),
    )(outp, bias.reshape(1, d), ln_g.reshape(1, d), ln_b.reshape(1, d))
    return out


# speedup vs baseline: 1.1451x; 1.1451x over previous
import functools

import jax
import jax.numpy as jnp
from jax import lax
from jax.experimental import pallas as pl
from jax.experimental.pallas import tpu as pltpu
from jax.experimental.pallas import tpu_sc as plsc

L = 16
C = 80
G = C // L

_PERM_DNUMS = lax.GatherDimensionNumbers(
    offset_dims=(), collapsed_slice_dims=(0,), start_index_map=(0,))


def _vperm(v, idx):
    return lax.gather(v, idx[:, None], _PERM_DNUMS, (1,),
                      mode=lax.GatherScatterMode.PROMISE_IN_BOUNDS)


def _prep_body(x_ref, wq_ref, w1_ref, qb_ref, q_ref, c_ref):
    x = x_ref[...]
    a = jnp.dot(wq_ref[...].T, w1_ref[...], preferred_element_type=jnp.float32)
    q = jnp.dot(x, a, preferred_element_type=jnp.float32)
    q = q + jnp.dot(qb_ref[...], w1_ref[...], preferred_element_type=jnp.float32)
    q_ref[...] = q
    d = x.shape[1]
    xn2 = jnp.sum(x * x, axis=1, keepdims=True)
    qn2 = jnp.sum(q * q, axis=1, keepdims=True)
    c_ref[...] = jnp.sqrt(qn2) * (jnp.sqrt(jnp.max(xn2)) / (d ** 0.5))


def _xw_body(x_ref, w_ref, o_ref):
    o_ref[0] = jnp.dot(x_ref[...], w_ref[0], preferred_element_type=jnp.float32)


def _rden_body(d_ref, r_ref):
    r_ref[...] = 1.0 / (d_ref[0:1, :] + d_ref[1:2, :])


def _final_body(p_ref, b_ref, g_ref, bb_ref, o_ref):
    o = p_ref[0] + p_ref[1] + b_ref[...]
    m = jnp.mean(o, axis=1, keepdims=True)
    ctr = o - m
    v = jnp.mean(ctr * ctr, axis=1, keepdims=True)
    o = ctr * jax.lax.rsqrt(v + 1e-5) * g_ref[...] + bb_ref[...]
    o_ref[...] = jnp.maximum(o, 0.0)


P1_SLOTS = 4


def _sc_pass1(n, e, rn, nc, ns):
    epw = e // (nc * ns)
    nchunks = epw // C
    seg_per_tile = rn // ns
    mesh = plsc.VectorSubcoreMesh(core_axis_name="c", subcore_axis_name="s")
    inv_scale = 1.0 / (128.0 ** 0.5)

    def body(src_h, dst_h, et_h, q_h, x_h, c_h, ex_h, den_h, *scr):
        srcb, dstb, etb, cloc, zbuf, scoreb = scr[0:6]
        qrows = scr[6:8]
        xrows = scr[8:10]
        srcc = scr[10:14]
        dstc = scr[14:18]
        segb = scr[18:22]
        exb = scr[22:26]
        den_sh = scr[26]
        gsem = scr[27:29]
        slin = scr[29:31]
        sind = scr[31:33]

        cid = lax.axis_index("c")
        sid = lax.axis_index("s")
        wid = cid * ns + sid
        base = wid * epw
        myseg = pl.ds(sid * seg_per_tile, seg_per_tile)

        def zf(i, carry):
            zbuf[pl.ds(i * L, L)] = jnp.zeros((L,), jnp.float32)
            return carry

        lax.fori_loop(0, seg_per_tile // L, zf, 0)
        pltpu.sync_copy(zbuf, den_sh.at[myseg])
        pltpu.sync_copy(src_h.at[pl.ds(base, epw)], srcb)
        pltpu.sync_copy(dst_h.at[pl.ds(base, epw)], dstb)
        pltpu.sync_copy(et_h.at[pl.ds(base, epw)], etb)
        pltpu.sync_copy(c_h, cloc)
        plsc.subcore_barrier()

        lane0 = lax.iota(jnp.int32, L) == 0
        lanes = lax.iota(jnp.int32, L)
        perms = [lanes ^ sh for sh in (8, 4, 2, 1)]

        def stage(ci, s):
            off = ci * C

            def stg(g, carry):
                sl = pl.ds(off + g * L, L)
                osl = pl.ds(g * L, L)
                d16 = dstb[sl]
                srcc[s][osl] = srcb[sl]
                dstc[s][osl] = d16
                segb[s][osl] = etb[sl] * n + d16
                return carry

            lax.fori_loop(0, G, stg, 0, unroll=True)

        def issue_gather(s, rb):
            pltpu.async_copy(q_h.at[dstc[s]], qrows[rb], gsem[rb])
            pltpu.async_copy(x_h.at[srcc[s]], xrows[rb], gsem[rb])

        def wait_gather(s, rb):
            pltpu.make_async_copy(q_h.at[dstc[s]], qrows[rb], gsem[rb]).wait()
            pltpu.make_async_copy(x_h.at[srcc[s]], xrows[rb], gsem[rb]).wait()

        def compute(s, rb):
            qr = qrows[rb]
            xr = xrows[rb]

            def dot(g, carry):
                osl = pl.ds(g * L, L)
                for j in range(L):
                    row = g * L + j
                    parts = [qr[row, pl.ds(k * L, L)] * xr[row, pl.ds(k * L, L)]
                             for k in range(8)]
                    while len(parts) > 1:
                        parts = [parts[i] + parts[i + 1]
                                 for i in range(0, len(parts), 2)]
                    acc = parts[0]
                    for p in perms:
                        acc = acc + _vperm(acc, p)
                    plsc.store_scatter(scoreb,
                                       [jnp.full((L,), row, jnp.int32)],
                                       acc, mask=lane0)
                c16 = plsc.load_gather(cloc, [dstc[s][osl]])
                exb[s][osl] = jnp.exp(scoreb[osl] * inv_scale - c16)
                return carry

            lax.fori_loop(0, G, dot, 0)

        def issue_stores(ci, s, rb):
            pltpu.async_copy(exb[s], ex_h.at[pl.ds(base + ci * C, C)],
                             slin[rb])
            pltpu.async_copy(exb[s], den_sh.at[segb[s]], sind[rb], add=True)

        def drain_stores(ci, s, rb):
            pltpu.make_async_copy(exb[s], ex_h.at[pl.ds(base + ci * C, C)],
                                  slin[rb]).wait()
            pltpu.make_async_copy(exb[s], den_sh.at[segb[s]],
                                  sind[rb]).wait()

        stage(0, 0)
        issue_gather(0, 0)
        stage(1, 1)
        issue_gather(1, 1)

        nb_outer = (nchunks + P1_SLOTS - 1) // P1_SLOTS

        def outer(h, carry):
            for b in range(P1_SLOTS):
                rb = b & 1
                s2 = (b + 2) % P1_SLOTS
                ci = h * P1_SLOTS + b

                @pl.when(ci < nchunks)
                def _():
                    wait_gather(b, rb)
                    compute(b, rb)

                    @pl.when(ci >= 2)
                    def _():
                        drain_stores(ci - 2, s2, rb)

                    issue_stores(ci, b, rb)

                    @pl.when(ci + 2 < nchunks)
                    def _():
                        stage(ci + 2, s2)
                        issue_gather(s2, rb)

            return carry

        lax.fori_loop(0, nb_outer, outer, 0)

        drain_stores(nchunks - 2, (nchunks - 2) % P1_SLOTS, (nchunks - 2) & 1)
        drain_stores(nchunks - 1, (nchunks - 1) % P1_SLOTS, (nchunks - 1) & 1)

        plsc.subcore_barrier()

        pltpu.sync_copy(den_sh.at[myseg], zbuf)
        pltpu.sync_copy(zbuf,
                        den_h.at[pl.ds(cid * rn + sid * seg_per_tile,
                                       seg_per_tile)])

    return pl.kernel(
        body,
        mesh=mesh,
        compiler_params=pltpu.CompilerParams(needs_layout_passes=False),
        out_type=[
            jax.ShapeDtypeStruct((e,), jnp.float32),
            jax.ShapeDtypeStruct((nc * rn,), jnp.float32),
        ],
        scratch_types=(
            [
                pltpu.VMEM((epw,), jnp.int32),
                pltpu.VMEM((epw,), jnp.int32),
                pltpu.VMEM((epw,), jnp.int32),
                pltpu.VMEM((n,), jnp.float32),
                pltpu.VMEM((rn // ns,), jnp.float32),
                pltpu.VMEM((C,), jnp.float32),
            ]
            + [pltpu.VMEM((C, 128), jnp.float32)] * 4
            + [pltpu.VMEM((C,), jnp.int32)] * (3 * P1_SLOTS)
            + [pltpu.VMEM((C,), jnp.float32)] * P1_SLOTS
            + [pltpu.VMEM_SHARED((rn,), jnp.float32)]
            + [pltpu.SemaphoreType.DMA] * 6
        ),
    )


P2_SLOTS = 6


def _sc_pass2(n, e, rn, nc, ns):
    epw = e // (nc * ns)
    nchunks = epw // C
    mesh = plsc.VectorSubcoreMesh(core_axis_name="c", subcore_axis_name="s")
    nblk = n // C

    def body(src_h, dst_h, et_h, ex_h, xw_h, rden_h, outp_h, *scr):
        wb = scr[0]
        rowsa = scr[1:3]
        rowsb = scr[3:5]
        srcc = scr[5:11]
        dstc = scr[11:17]
        etc_ = scr[17:23]
        excb = scr[23:29]
        gsrcc = scr[29:35]
        segc = scr[35:41]
        rdenc = scr[41:47]
        out_sh = scr[47]
        gsem = scr[48:50]
        scsem = scr[50:52]
        lsem = scr[52:58]

        cid = lax.axis_index("c")
        sid = lax.axis_index("s")
        wid = cid * ns + sid
        base = wid * epw

        def zrow(i, carry):
            for k in range(8):
                rowsa[0][i, pl.ds(k * L, L)] = jnp.zeros((L,), jnp.float32)
            return carry

        lax.fori_loop(0, C, zrow, 0)

        def zblk(i, carry):
            blk = sid + i * ns

            @pl.when(blk < nblk)
            def _():
                pltpu.sync_copy(rowsa[0], out_sh.at[pl.ds(blk * C, C)])

            return carry

        lax.fori_loop(0, (nblk + ns - 1) // ns, zblk, 0)
        plsc.subcore_barrier()

        def issue_loads(ci, s):
            gb = base + ci * C
            pltpu.async_copy(src_h.at[pl.ds(gb, C)], srcc[s], lsem[s])
            pltpu.async_copy(dst_h.at[pl.ds(gb, C)], dstc[s], lsem[s])
            pltpu.async_copy(et_h.at[pl.ds(gb, C)], etc_[s], lsem[s])
            pltpu.async_copy(ex_h.at[pl.ds(gb, C)], excb[s], lsem[s])

        def drain_loads(ci, s):
            gb = base + ci * C
            pltpu.make_async_copy(src_h.at[pl.ds(gb, C)], srcc[s],
                                  lsem[s]).wait()
            pltpu.make_async_copy(dst_h.at[pl.ds(gb, C)], dstc[s],
                                  lsem[s]).wait()
            pltpu.make_async_copy(et_h.at[pl.ds(gb, C)], etc_[s],
                                  lsem[s]).wait()
            pltpu.make_async_copy(ex_h.at[pl.ds(gb, C)], excb[s],
                                  lsem[s]).wait()

        def stage(s):
            def stg(g, carry):
                osl = pl.ds(g * L, L)
                gsrcc[s][osl] = etc_[s][osl] * n + srcc[s][osl]
                segc[s][osl] = etc_[s][osl] * n + dstc[s][osl]
                return carry

            lax.fori_loop(0, G, stg, 0, unroll=True)

        def issue_gather(s, rb):
            pltpu.async_copy(xw_h.at[gsrcc[s]], rowsa[rb], gsem[rb])
            pltpu.async_copy(rden_h.at[segc[s]], rdenc[s], gsem[rb])

        def wait_gather(s, rb):
            pltpu.make_async_copy(xw_h.at[gsrcc[s]], rowsa[rb],
                                  gsem[rb]).wait()
            pltpu.make_async_copy(rden_h.at[segc[s]], rdenc[s],
                                  gsem[rb]).wait()

        def compute(s, rb):
            ra = rowsa[rb]
            rb_ = rowsb[rb]

            def scale(g, carry):
                osl = pl.ds(g * L, L)
                wb[osl] = excb[s][osl] * rdenc[s][osl]
                for j in range(L):
                    row = g * L + j
                    wj = plsc.load_gather(wb, [jnp.full((L,), row, jnp.int32)])
                    for k in range(8):
                        sl2 = pl.ds(k * L, L)
                        rb_[row, sl2] = ra[row, sl2] * wj
                return carry

            lax.fori_loop(0, G, scale, 0)

        def issue_scatter(s, rb):
            pltpu.async_copy(rowsb[rb], out_sh.at[dstc[s]], scsem[rb],
                             add=True)

        def drain_scatter(s, rb):
            pltpu.make_async_copy(rowsb[rb], out_sh.at[dstc[s]],
                                  scsem[rb]).wait()

        for s in range(4):
            issue_loads(s, s)
        drain_loads(0, 0)
        stage(0)
        issue_gather(0, 0)
        drain_loads(1, 1)
        stage(1)
        issue_gather(1, 1)

        nb_outer = (nchunks + P2_SLOTS - 1) // P2_SLOTS

        def outer(h, carry):
            for b in range(P2_SLOTS):
                rb = b & 1
                s2 = (b + 2) % P2_SLOTS
                s4 = (b + 4) % P2_SLOTS
                ci = h * P2_SLOTS + b

                @pl.when(ci < nchunks)
                def _():
                    wait_gather(b, rb)

                    @pl.when(ci >= 2)
                    def _():
                        drain_scatter(s4, rb)

                    compute(b, rb)
                    issue_scatter(b, rb)

                    @pl.when(ci + 4 < nchunks)
                    def _():
                        issue_loads(ci + 4, s4)

                    @pl.when(ci + 2 < nchunks)
                    def _():
                        drain_loads(ci + 2, s2)
                        stage(s2)
                        issue_gather(s2, rb)

            return carry

        lax.fori_loop(0, nb_outer, outer, 0)

        drain_scatter((nchunks - 2) % P2_SLOTS, (nchunks - 2) & 1)
        drain_scatter((nchunks - 1) % P2_SLOTS, (nchunks - 1) & 1)

        plsc.subcore_barrier()

        def dblk(i, carry):
            blk = sid + i * ns

            @pl.when(blk < nblk)
            def _():
                pltpu.sync_copy(out_sh.at[pl.ds(blk * C, C)], rowsa[0])
                pltpu.sync_copy(rowsa[0], outp_h.at[cid, pl.ds(blk * C, C)])

            return carry

        lax.fori_loop(0, (nblk + ns - 1) // ns, dblk, 0)

    return pl.kernel(
        body,
        mesh=mesh,
        compiler_params=pltpu.CompilerParams(needs_layout_passes=False),
        out_type=jax.ShapeDtypeStruct((nc, n, 128), jnp.float32),
        scratch_types=(
            [pltpu.VMEM((C,), jnp.float32)]
            + [pltpu.VMEM((C, 128), jnp.float32)] * 4
            + [pltpu.VMEM((C,), jnp.int32)] * (3 * P2_SLOTS)
            + [pltpu.VMEM((C,), jnp.float32)] * P2_SLOTS
            + [pltpu.VMEM((C,), jnp.int32)] * (2 * P2_SLOTS)
            + [pltpu.VMEM((C,), jnp.float32)] * P2_SLOTS
            + [pltpu.VMEM_SHARED((n, 128), jnp.float32)]
            + [pltpu.SemaphoreType.DMA] * (4 + P2_SLOTS)
        ),
    )


@jax.jit
def kernel(x, edge_index, edge_type, rule_ids, W_r, Wq_w, Wq_b, Wk_w, Wk_b,
           rule_emb, bias, ln_g, ln_b):
    n, d = x.shape
    e = edge_type.shape[0]
    r = W_r.shape[0]
    rn = r * n
    info = plsc.get_sparse_core_info()
    nc, ns = info.num_cores, info.num_subcores

    src = edge_index[0]
    dst = edge_index[1]
    et = edge_type.astype(jnp.int32)
    w1 = Wk_w[:, :d]

    q, c2 = pl.pallas_call(
        _prep_body,
        out_shape=[
            jax.ShapeDtypeStruct((n, d), jnp.float32),
            jax.ShapeDtypeStruct((n, 1), jnp.float32),
        ],
    )(x, Wq_w, w1, Wq_b.reshape(1, d))
    c = c2.reshape(n)

    xw = pl.pallas_call(
        _xw_body,
        grid=(r,),
        in_specs=[
            pl.BlockSpec((n, d), lambda i: (0, 0)),
            pl.BlockSpec((1, d, d), lambda i: (i, 0, 0)),
        ],
        out_specs=pl.BlockSpec((1, n, d), lambda i: (i, 0, 0)),
        out_shape=jax.ShapeDtypeStruct((r, n, d), jnp.float32),
    )(x, W_r).reshape(rn, d)

    ex, den = _sc_pass1(n, e, rn, nc, ns)(src, dst, et, q, x, c)

    rden = pl.pallas_call(
        _rden_body,
        out_shape=jax.ShapeDtypeStruct((1, rn), jnp.float32),
    )(den.reshape(nc, rn)).reshape(rn)

    outp = _sc_pass2(n, e, rn, nc, ns)(src, dst, et, ex, xw, rden)

    out = pl.pallas_call(
        _final_body,
        out_shape=jax.ShapeDtypeStruct((n, d), jnp.float32),
    )(outp, bias.reshape(1, d), ln_g.reshape(1, d), ln_b.reshape(1, d))
    return out


# speedup vs baseline: 1.2126x; 1.0590x over previous
import functools

import jax
import jax.numpy as jnp
from jax import lax
from jax.experimental import pallas as pl
from jax.experimental.pallas import tpu as pltpu
from jax.experimental.pallas import tpu_sc as plsc

L = 16
C = 80
G = C // L

_PERM_DNUMS = lax.GatherDimensionNumbers(
    offset_dims=(), collapsed_slice_dims=(0,), start_index_map=(0,))


def _vperm(v, idx):
    return lax.gather(v, idx[:, None], _PERM_DNUMS, (1,),
                      mode=lax.GatherScatterMode.PROMISE_IN_BOUNDS)


def _prep_body(x_ref, wq_ref, w1_ref, qb_ref, q_ref, c_ref):
    x = x_ref[...]
    a = jnp.dot(wq_ref[...].T, w1_ref[...], preferred_element_type=jnp.float32)
    q = jnp.dot(x, a, preferred_element_type=jnp.float32)
    q = q + jnp.dot(qb_ref[...], w1_ref[...], preferred_element_type=jnp.float32)
    q_ref[...] = q
    d = x.shape[1]
    xn2 = jnp.sum(x * x, axis=1, keepdims=True)
    qn2 = jnp.sum(q * q, axis=1, keepdims=True)
    c_ref[...] = jnp.sqrt(qn2) * (jnp.sqrt(jnp.max(xn2)) / (d ** 0.5))


def _xw_body(x_ref, w_ref, o_ref):
    o_ref[0] = jnp.dot(x_ref[...], w_ref[0], preferred_element_type=jnp.float32)


def _final_body(p_ref, b_ref, g_ref, bb_ref, o_ref):
    o = p_ref[0] + p_ref[1] + b_ref[...]
    m = jnp.mean(o, axis=1, keepdims=True)
    ctr = o - m
    v = jnp.mean(ctr * ctr, axis=1, keepdims=True)
    o = ctr * jax.lax.rsqrt(v + 1e-5) * g_ref[...] + bb_ref[...]
    o_ref[...] = jnp.maximum(o, 0.0)


P1_SLOTS = 4


def _sc_pass1(n, e, rn, nc, ns):
    epw = e // (nc * ns)
    nchunks = epw // C
    seg_per_tile = rn // ns
    mesh = plsc.VectorSubcoreMesh(core_axis_name="c", subcore_axis_name="s")
    inv_scale = 1.0 / (128.0 ** 0.5)

    def body(src_h, dst_h, et_h, q_h, x_h, c_h, ex_h, den_h, *scr):
        srcb, dstb, etb, cloc, zbuf, scoreb = scr[0:6]
        qrows = scr[6:8]
        xrows = scr[8:10]
        srcc = scr[10:14]
        dstc = scr[14:18]
        segb = scr[18:22]
        exb = scr[22:26]
        den_sh = scr[26]
        gsem = scr[27:29]
        slin = scr[29:31]
        sind = scr[31:33]

        cid = lax.axis_index("c")
        sid = lax.axis_index("s")
        wid = cid * ns + sid
        base = wid * epw
        myseg = pl.ds(sid * seg_per_tile, seg_per_tile)

        def zf(i, carry):
            zbuf[pl.ds(i * L, L)] = jnp.zeros((L,), jnp.float32)
            return carry

        lax.fori_loop(0, seg_per_tile // L, zf, 0)
        pltpu.sync_copy(zbuf, den_sh.at[myseg])
        pltpu.sync_copy(src_h.at[pl.ds(base, epw)], srcb)
        pltpu.sync_copy(dst_h.at[pl.ds(base, epw)], dstb)
        pltpu.sync_copy(et_h.at[pl.ds(base, epw)], etb)
        pltpu.sync_copy(c_h, cloc)
        plsc.subcore_barrier()

        lane0 = lax.iota(jnp.int32, L) == 0
        lanes = lax.iota(jnp.int32, L)
        perms = [lanes ^ sh for sh in (8, 4, 2, 1)]

        def stage(ci, s):
            off = ci * C

            def stg(g, carry):
                sl = pl.ds(off + g * L, L)
                osl = pl.ds(g * L, L)
                d16 = dstb[sl]
                srcc[s][osl] = srcb[sl]
                dstc[s][osl] = d16
                segb[s][osl] = etb[sl] * n + d16
                return carry

            lax.fori_loop(0, G, stg, 0, unroll=True)

        def issue_gather(s, rb):
            pltpu.async_copy(q_h.at[dstc[s]], qrows[rb], gsem[rb])
            pltpu.async_copy(x_h.at[srcc[s]], xrows[rb], gsem[rb])

        def wait_gather(s, rb):
            pltpu.make_async_copy(q_h.at[dstc[s]], qrows[rb], gsem[rb]).wait()
            pltpu.make_async_copy(x_h.at[srcc[s]], xrows[rb], gsem[rb]).wait()

        def compute(s, rb):
            qr = qrows[rb]
            xr = xrows[rb]

            def dot(g, carry):
                osl = pl.ds(g * L, L)
                for j in range(L):
                    row = g * L + j
                    acc = qr[row, pl.ds(0, L)] * xr[row, pl.ds(0, L)]
                    for k in range(1, 8):
                        sl2 = pl.ds(k * L, L)
                        acc = acc + qr[row, sl2] * xr[row, sl2]
                    for p in perms:
                        acc = acc + _vperm(acc, p)
                    plsc.store_scatter(scoreb,
                                       [jnp.full((L,), row, jnp.int32)],
                                       acc, mask=lane0)
                c16 = plsc.load_gather(cloc, [dstc[s][osl]])
                exb[s][osl] = jnp.exp(scoreb[osl] * inv_scale - c16)
                return carry

            lax.fori_loop(0, G, dot, 0)

        def issue_stores(ci, s, rb):
            pltpu.async_copy(exb[s], ex_h.at[pl.ds(base + ci * C, C)],
                             slin[rb])
            pltpu.async_copy(exb[s], den_sh.at[segb[s]], sind[rb], add=True)

        def drain_stores(ci, s, rb):
            pltpu.make_async_copy(exb[s], ex_h.at[pl.ds(base + ci * C, C)],
                                  slin[rb]).wait()
            pltpu.make_async_copy(exb[s], den_sh.at[segb[s]],
                                  sind[rb]).wait()

        stage(0, 0)
        issue_gather(0, 0)
        stage(1, 1)
        issue_gather(1, 1)

        nb_outer = (nchunks + P1_SLOTS - 1) // P1_SLOTS

        def outer(h, carry):
            for b in range(P1_SLOTS):
                rb = b & 1
                s2 = (b + 2) % P1_SLOTS
                ci = h * P1_SLOTS + b

                @pl.when(ci < nchunks)
                def _():
                    wait_gather(b, rb)
                    compute(b, rb)

                    @pl.when(ci >= 2)
                    def _():
                        drain_stores(ci - 2, s2, rb)

                    issue_stores(ci, b, rb)

                    @pl.when(ci + 2 < nchunks)
                    def _():
                        stage(ci + 2, s2)
                        issue_gather(s2, rb)

            return carry

        lax.fori_loop(0, nb_outer, outer, 0)

        drain_stores(nchunks - 2, (nchunks - 2) % P1_SLOTS, (nchunks - 2) & 1)
        drain_stores(nchunks - 1, (nchunks - 1) % P1_SLOTS, (nchunks - 1) & 1)

        plsc.subcore_barrier()

        pltpu.sync_copy(den_sh.at[myseg], zbuf)
        pltpu.sync_copy(zbuf,
                        den_h.at[pl.ds(cid * rn + sid * seg_per_tile,
                                       seg_per_tile)])

    return pl.kernel(
        body,
        mesh=mesh,
        compiler_params=pltpu.CompilerParams(needs_layout_passes=False),
        out_type=[
            jax.ShapeDtypeStruct((e,), jnp.float32),
            jax.ShapeDtypeStruct((nc * rn,), jnp.float32),
        ],
        scratch_types=(
            [
                pltpu.VMEM((epw,), jnp.int32),
                pltpu.VMEM((epw,), jnp.int32),
                pltpu.VMEM((epw,), jnp.int32),
                pltpu.VMEM((n,), jnp.float32),
                pltpu.VMEM((rn // ns,), jnp.float32),
                pltpu.VMEM((C,), jnp.float32),
            ]
            + [pltpu.VMEM((C, 128), jnp.float32)] * 4
            + [pltpu.VMEM((C,), jnp.int32)] * (3 * P1_SLOTS)
            + [pltpu.VMEM((C,), jnp.float32)] * P1_SLOTS
            + [pltpu.VMEM_SHARED((rn,), jnp.float32)]
            + [pltpu.SemaphoreType.DMA] * 6
        ),
    )


P2_SLOTS = 6


def _sc_pass2(n, e, rn, nc, ns):
    epw = e // (nc * ns)
    nchunks = epw // C
    mesh = plsc.VectorSubcoreMesh(core_axis_name="c", subcore_axis_name="s")
    nblk = n // C

    def body(src_h, dst_h, et_h, ex_h, xw_h, den2_h, outp_h, *scr):
        wb = scr[0]
        rowsa = scr[1:3]
        rowsb = scr[3:5]
        srcc = scr[5:11]
        dstc = scr[11:17]
        etc_ = scr[17:23]
        excb = scr[23:29]
        gsrcc = scr[29:35]
        segc = scr[35:41]
        seg2c = scr[41:47]
        d0c = scr[47:53]
        d1c = scr[53:59]
        out_sh = scr[59]
        gsem = scr[60:62]
        scsem = scr[62:64]
        lsem = scr[64:70]
        dsem = scr[70:72]

        cid = lax.axis_index("c")
        sid = lax.axis_index("s")
        wid = cid * ns + sid
        base = wid * epw

        def zrow(i, carry):
            for k in range(8):
                rowsa[0][i, pl.ds(k * L, L)] = jnp.zeros((L,), jnp.float32)
            return carry

        lax.fori_loop(0, C, zrow, 0)

        def zblk(i, carry):
            blk = sid + i * ns

            @pl.when(blk < nblk)
            def _():
                pltpu.sync_copy(rowsa[0], out_sh.at[pl.ds(blk * C, C)])

            return carry

        lax.fori_loop(0, (nblk + ns - 1) // ns, zblk, 0)
        plsc.subcore_barrier()

        def issue_loads(ci, s):
            gb = base + ci * C
            pltpu.async_copy(src_h.at[pl.ds(gb, C)], srcc[s], lsem[s])
            pltpu.async_copy(dst_h.at[pl.ds(gb, C)], dstc[s], lsem[s])
            pltpu.async_copy(et_h.at[pl.ds(gb, C)], etc_[s], lsem[s])
            pltpu.async_copy(ex_h.at[pl.ds(gb, C)], excb[s], lsem[s])

        def drain_loads(ci, s):
            gb = base + ci * C
            pltpu.make_async_copy(src_h.at[pl.ds(gb, C)], srcc[s],
                                  lsem[s]).wait()
            pltpu.make_async_copy(dst_h.at[pl.ds(gb, C)], dstc[s],
                                  lsem[s]).wait()
            pltpu.make_async_copy(et_h.at[pl.ds(gb, C)], etc_[s],
                                  lsem[s]).wait()
            pltpu.make_async_copy(ex_h.at[pl.ds(gb, C)], excb[s],
                                  lsem[s]).wait()

        def stage(s):
            def stg(g, carry):
                osl = pl.ds(g * L, L)
                seg = etc_[s][osl] * n + dstc[s][osl]
                gsrcc[s][osl] = etc_[s][osl] * n + srcc[s][osl]
                segc[s][osl] = seg
                seg2c[s][osl] = seg + rn
                return carry

            lax.fori_loop(0, G, stg, 0, unroll=True)

        def issue_gather(s, rb):
            pltpu.async_copy(xw_h.at[gsrcc[s]], rowsa[rb], gsem[rb])
            pltpu.async_copy(den2_h.at[segc[s]], d0c[s], dsem[rb])
            pltpu.async_copy(den2_h.at[seg2c[s]], d1c[s], dsem[rb])

        def wait_gather(s, rb):
            pltpu.make_async_copy(xw_h.at[gsrcc[s]], rowsa[rb],
                                  gsem[rb]).wait()
            pltpu.make_async_copy(den2_h.at[segc[s]], d0c[s],
                                  dsem[rb]).wait()
            pltpu.make_async_copy(den2_h.at[seg2c[s]], d1c[s],
                                  dsem[rb]).wait()

        def compute(s, rb):
            ra = rowsa[rb]
            rb_ = rowsb[rb]

            def scale(g, carry):
                osl = pl.ds(g * L, L)
                wb[osl] = excb[s][osl] / (d0c[s][osl] + d1c[s][osl])
                for j in range(L):
                    row = g * L + j
                    wj = plsc.load_gather(wb, [jnp.full((L,), row, jnp.int32)])
                    for k in range(8):
                        sl2 = pl.ds(k * L, L)
                        rb_[row, sl2] = ra[row, sl2] * wj
                return carry

            lax.fori_loop(0, G, scale, 0)

        def issue_scatter(s, rb):
            pltpu.async_copy(rowsb[rb], out_sh.at[dstc[s]], scsem[rb],
                             add=True)

        def drain_scatter(s, rb):
            pltpu.make_async_copy(rowsb[rb], out_sh.at[dstc[s]],
                                  scsem[rb]).wait()

        for s in range(4):
            issue_loads(s, s)
        drain_loads(0, 0)
        stage(0)
        issue_gather(0, 0)
        drain_loads(1, 1)
        stage(1)
        issue_gather(1, 1)

        nb_outer = (nchunks + P2_SLOTS - 1) // P2_SLOTS

        def outer(h, carry):
            for b in range(P2_SLOTS):
                rb = b & 1
                s2 = (b + 2) % P2_SLOTS
                s4 = (b + 4) % P2_SLOTS
                ci = h * P2_SLOTS + b

                @pl.when(ci < nchunks)
                def _():
                    wait_gather(b, rb)

                    @pl.when(ci >= 2)
                    def _():
                        drain_scatter(s4, rb)

                    compute(b, rb)
                    issue_scatter(b, rb)

                    @pl.when(ci + 4 < nchunks)
                    def _():
                        issue_loads(ci + 4, s4)

                    @pl.when(ci + 2 < nchunks)
                    def _():
                        drain_loads(ci + 2, s2)
                        stage(s2)
                        issue_gather(s2, rb)

            return carry

        lax.fori_loop(0, nb_outer, outer, 0)

        drain_scatter((nchunks - 2) % P2_SLOTS, (nchunks - 2) & 1)
        drain_scatter((nchunks - 1) % P2_SLOTS, (nchunks - 1) & 1)

        plsc.subcore_barrier()

        def dblk(i, carry):
            blk = sid + i * ns

            @pl.when(blk < nblk)
            def _():
                pltpu.sync_copy(out_sh.at[pl.ds(blk * C, C)], rowsa[0])
                pltpu.sync_copy(rowsa[0], outp_h.at[cid, pl.ds(blk * C, C)])

            return carry

        lax.fori_loop(0, (nblk + ns - 1) // ns, dblk, 0)

    return pl.kernel(
        body,
        mesh=mesh,
        compiler_params=pltpu.CompilerParams(needs_layout_passes=False),
        out_type=jax.ShapeDtypeStruct((nc, n, 128), jnp.float32),
        scratch_types=(
            [pltpu.VMEM((C,), jnp.float32)]
            + [pltpu.VMEM((C, 128), jnp.float32)] * 4
            + [pltpu.VMEM((C,), jnp.int32)] * (3 * P2_SLOTS)
            + [pltpu.VMEM((C,), jnp.float32)] * P2_SLOTS
            + [pltpu.VMEM((C,), jnp.int32)] * (3 * P2_SLOTS)
            + [pltpu.VMEM((C,), jnp.float32)] * (2 * P2_SLOTS)
            + [pltpu.VMEM_SHARED((n, 128), jnp.float32)]
            + [pltpu.SemaphoreType.DMA] * (6 + P2_SLOTS)
        ),
    )


@jax.jit
def kernel(x, edge_index, edge_type, rule_ids, W_r, Wq_w, Wq_b, Wk_w, Wk_b,
           rule_emb, bias, ln_g, ln_b):
    n, d = x.shape
    e = edge_type.shape[0]
    r = W_r.shape[0]
    rn = r * n
    info = plsc.get_sparse_core_info()
    nc, ns = info.num_cores, info.num_subcores

    src = edge_index[0]
    dst = edge_index[1]
    et = edge_type.astype(jnp.int32)
    w1 = Wk_w[:, :d]

    q, c2 = pl.pallas_call(
        _prep_body,
        out_shape=[
            jax.ShapeDtypeStruct((n, d), jnp.float32),
            jax.ShapeDtypeStruct((n, 1), jnp.float32),
        ],
    )(x, Wq_w, w1, Wq_b.reshape(1, d))
    c = c2.reshape(n)

    xw = pl.pallas_call(
        _xw_body,
        grid=(r,),
        in_specs=[
            pl.BlockSpec((n, d), lambda i: (0, 0)),
            pl.BlockSpec((1, d, d), lambda i: (i, 0, 0)),
        ],
        out_specs=pl.BlockSpec((1, n, d), lambda i: (i, 0, 0)),
        out_shape=jax.ShapeDtypeStruct((r, n, d), jnp.float32),
    )(x, W_r).reshape(rn, d)

    ex, den = _sc_pass1(n, e, rn, nc, ns)(src, dst, et, q, x, c)

    outp = _sc_pass2(n, e, rn, nc, ns)(src, dst, et, ex, xw, den)

    out = pl.pallas_call(
        _final_body,
        out_shape=jax.ShapeDtypeStruct((n, d), jnp.float32),
    )(outp, bias.reshape(1, d), ln_g.reshape(1, d), ln_b.reshape(1, d))
    return out
